# baseline (speedup 1.0000x reference)
import jax
import jax.numpy as jnp
import numpy as np
from jax.experimental import pallas as pl

_B, _L, _C, _NS = 1, 8192, 8, 64
_LANES = 1024
_ROWS = _NS * _B * _L * _C // _LANES
_BLK = 64

_KS0 = np.uint32(0)
_KS1 = np.uint32(1234)
_KS2 = np.uint32(_KS0 ^ _KS1 ^ np.uint32(0x1BD11BDA))
_TINY = np.float32(np.finfo(np.float32).tiny)
_ROT_A = (13, 15, 26, 6)
_ROT_B = (17, 29, 16, 24)


def _rotl(x, r):
    return (x << np.uint32(r)) | (x >> np.uint32(32 - r))


def _threefry_rounds(x0, x1, rots):
    for r in rots:
        x0 = x0 + x1
        x1 = _rotl(x1, r)
        x1 = x0 ^ x1
    return x0, x1


def _threefry_bits(idx):
    x0 = jnp.full(idx.shape, _KS0, jnp.uint32)
    x1 = idx + _KS1
    x0, x1 = _threefry_rounds(x0, x1, _ROT_A)
    x0, x1 = x0 + _KS1, x1 + (_KS2 + np.uint32(1))
    x0, x1 = _threefry_rounds(x0, x1, _ROT_B)
    x0, x1 = x0 + _KS2, x1 + (_KS0 + np.uint32(2))
    x0, x1 = _threefry_rounds(x0, x1, _ROT_A)
    x0, x1 = x0 + _KS0, x1 + (_KS1 + np.uint32(3))
    x0, x1 = _threefry_rounds(x0, x1, _ROT_B)
    x0, x1 = x0 + _KS1, x1 + (_KS2 + np.uint32(4))
    x0, x1 = _threefry_rounds(x0, x1, _ROT_A)
    x0, x1 = x0 + _KS2, x1 + (_KS0 + np.uint32(5))
    return x0 ^ x1


def _sample_kernel(lg_ref, out_ref):
    n = pl.program_id(0)
    shape = (_BLK, _LANES)
    row = jax.lax.broadcasted_iota(jnp.uint32, shape, 0)
    lane = jax.lax.broadcasted_iota(jnp.uint32, shape, 1)
    base = (n * (_BLK * _LANES)).astype(jnp.uint32)
    idx = base + row * np.uint32(_LANES) + lane

    bits = _threefry_bits(idx)
    fbits = (bits >> np.uint32(9)) | np.uint32(0x3F800000)
    floats = jax.lax.bitcast_convert_type(fbits, jnp.float32) - np.float32(1.0)
    u = jnp.maximum(_TINY, floats * (np.float32(1.0) - _TINY) + _TINY)
    g = -jnp.log(-jnp.log(u))

    c = lane & np.uint32(7)
    active = (c >= np.uint32(2)) & (c <= np.uint32(5))
    masked = jnp.where(active, lg_ref[...], np.float32(-10000.0))
    s = g + masked

    m = s
    for k in (1, 2, 4):
        fwd = jnp.roll(m, -k, axis=1)
        bwd = jnp.roll(m, k, axis=1)
        m = jnp.maximum(m, jnp.where((lane & np.uint32(k)) == 0, fwd, bwd))

    out_ref[...] = jnp.where(s == m, np.float32(1.0), np.float32(0.0))


def kernel(logits):
    lg = logits.reshape(_L * _C // _LANES, _LANES)
    out = pl.pallas_call(
        _sample_kernel,
        grid=(_ROWS // _BLK,),
        in_specs=[pl.BlockSpec((_L * _C // _LANES, _LANES), lambda i: (0, 0))],
        out_specs=pl.BlockSpec((_BLK, _LANES), lambda i: (i, 0)),
        out_shape=jax.ShapeDtypeStruct((_ROWS, _LANES), jnp.float32),
    )(lg)
    return out.reshape(_B, _NS, _L, _C)
